# baseline (speedup 1.0000x reference)
import functools

import numpy as np

import jax
import jax.numpy as jnp
from jax import lax
from jax.experimental import pallas as pl
from jax.experimental.pallas import tpu as pltpu
from jax.experimental.pallas import tpu_sc as plsc

_VOCAB = 100000
_NC = 2
_NS = 16
_PER = 3136
_LAST = _VOCAB - 31 * _PER
_L = 16


def _gumbel_table() -> np.ndarray:
    old = np.seterr(over="ignore")
    try:
        idx = np.arange(_VOCAB, dtype=np.uint64)
        x0 = (idx >> np.uint64(32)).astype(np.uint32)
        x1 = (idx & np.uint64(0xFFFFFFFF)).astype(np.uint32)
        k0, k1 = np.uint32(0), np.uint32(42)
        ks = [k0, k1, k0 ^ k1 ^ np.uint32(0x1BD11BDA)]

        def rotl(v, d):
            return (v << np.uint32(d)) | (v >> np.uint32(32 - d))

        x = [x0 + ks[0], x1 + ks[1]]

        def four_rounds(x, rots):
            for r in rots:
                x[0] = x[0] + x[1]
                x[1] = x[0] ^ rotl(x[1], r)
            return x

        ra, rb = (13, 15, 26, 6), (17, 29, 16, 24)
        x = four_rounds(x, ra); x[0] += ks[1]; x[1] += ks[2] + np.uint32(1)
        x = four_rounds(x, rb); x[0] += ks[2]; x[1] += ks[0] + np.uint32(2)
        x = four_rounds(x, ra); x[0] += ks[0]; x[1] += ks[1] + np.uint32(3)
        x = four_rounds(x, rb); x[0] += ks[1]; x[1] += ks[2] + np.uint32(4)
        x = four_rounds(x, ra); x[0] += ks[2]; x[1] += ks[0] + np.uint32(5)
        bits = x[0] ^ x[1]

        fb = (bits >> np.uint32(9)) | np.float32(1.0).view(np.uint32)
        f = fb.view(np.float32) - np.float32(1.0)
        tiny = np.float32(np.finfo(np.float32).tiny)
        u = np.maximum(tiny, f * (np.float32(1.0) - tiny) + tiny)
        return (-np.log(-np.log(u))).astype(np.float32)
    finally:
        np.seterr(**old)


_GUMBEL = _gumbel_table()

_mesh = plsc.VectorSubcoreMesh(
    core_axis_name="c", subcore_axis_name="s", num_cores=_NC)


@functools.partial(
    pl.kernel,
    out_type=jax.ShapeDtypeStruct((_NC * _L,), jnp.float32),
    mesh=_mesh,
    scratch_types=[
        pltpu.VMEM((_PER,), jnp.float32),
        pltpu.VMEM((_PER,), jnp.float32),
        pltpu.VMEM((_L,), jnp.float32),
        pltpu.VMEM((_NS * _L,), jnp.float32),
        pltpu.VMEM_SHARED((_NS * _L,), jnp.float32),
        pltpu.SemaphoreType.DMA,
    ],
    compiler_params=pltpu.CompilerParams(
        needs_layout_passes=False, skip_device_barrier=True),
)
def _sc_sample(x_hbm, g_hbm, out_hbm, xv, gv, stage, red, sh, sem):
    cid = lax.axis_index("c")
    sid = lax.axis_index("s")
    tid = cid * _NS + sid
    is_last = tid == _NC * _NS - 1
    base = tid * _PER
    nv = jnp.where(is_last, _LAST // _L, _PER // _L)

    @pl.when(is_last)
    def _():
        g_copy = pltpu.async_copy(g_hbm.at[pl.ds(base, _LAST)],
                                  gv.at[pl.ds(0, _LAST)], sem)
        pltpu.sync_copy(x_hbm.at[pl.ds(base, _LAST)], xv.at[pl.ds(0, _LAST)])
        g_copy.wait()

    @pl.when(jnp.logical_not(is_last))
    def _():
        g_copy = pltpu.async_copy(g_hbm.at[pl.ds(base, _PER)], gv, sem)
        pltpu.sync_copy(x_hbm.at[pl.ds(base, _PER)], xv)
        g_copy.wait()

    lane = lax.iota(jnp.int32, _L)
    ninf = jnp.full((_L,), -jnp.inf, jnp.float32)

    @plsc.parallel_loop(
        0, nv, unroll=8,
        carry=(ninf, ninf, jnp.full((_L,), 0, jnp.int32)))
    def _sample(i, carry):
        m16, vb, ib = carry
        off = i * _L
        xs = xv[pl.ds(off, _L)]
        v = xs + gv[pl.ds(off, _L)]
        upd = v > vb
        return (jnp.maximum(m16, xs),
                jnp.where(upd, v, vb),
                jnp.where(upd, lane + (base + off), ib))

    m16, vb, ib = _sample
    m_t = jnp.max(m16)
    best = jnp.max(vb)
    bidx = jnp.min(jnp.where(vb == best, ib, jnp.int32(2**31 - 1)))

    packed = jnp.where(lane == 0, best - m_t,
                       jnp.where(lane == 1, plsc.bitcast(
                           jnp.full((_L,), bidx, jnp.int32), jnp.float32),
                           jnp.full((_L,), m_t, jnp.float32)))
    stage[...] = packed
    pltpu.sync_copy(stage, sh.at[pl.ds(sid * _L, _L)])
    plsc.subcore_barrier()

    @pl.when(sid == 0)
    def _():
        pltpu.sync_copy(sh, red)

        def fbody(r, carry):
            cb, cr = carry
            row = red[pl.ds(r * _L, _L)]
            score = jnp.max(jnp.where(lane == 0, row, ninf)) \
                + jnp.max(jnp.where(lane == 2, row, ninf))
            take = score > cb
            return (jnp.where(take, score, cb),
                    jnp.where(jnp.full((_L,), take, jnp.bool_), row, cr))

        _, frow = lax.fori_loop(0, _NS, fbody,
                                (jnp.float32(-jnp.inf), ninf))
        stage[...] = frow
        pltpu.sync_copy(stage, out_hbm.at[pl.ds(cid * _L, _L)])


def kernel(outputs):
    x = outputs.reshape(_VOCAB)
    g = jnp.asarray(_GUMBEL)
    w = _sc_sample(x, g)
    s0 = w[0] + w[2]
    s1 = w[_L] + w[_L + 2]
    i0 = lax.bitcast_convert_type(w[1], jnp.int32)
    i1 = lax.bitcast_convert_type(w[_L + 1], jnp.int32)
    win = jnp.where(s1 > s0, i1, i0)
    return win.reshape(1, 1).astype(jnp.int64)


# speedup vs baseline: 1.3139x; 1.3139x over previous
import functools

import numpy as np

import jax
import jax.numpy as jnp
from jax import lax
from jax.experimental import pallas as pl
from jax.experimental.pallas import tpu as pltpu
from jax.experimental.pallas import tpu_sc as plsc

_VOCAB = 100000
_NT = 16
_PER = 6272
_LAST = _VOCAB - 15 * _PER
_L = 16


def _gumbel_table() -> np.ndarray:
    old = np.seterr(over="ignore")
    try:
        idx = np.arange(_VOCAB, dtype=np.uint64)
        x0 = (idx >> np.uint64(32)).astype(np.uint32)
        x1 = (idx & np.uint64(0xFFFFFFFF)).astype(np.uint32)
        k0, k1 = np.uint32(0), np.uint32(42)
        ks = [k0, k1, k0 ^ k1 ^ np.uint32(0x1BD11BDA)]

        def rotl(v, d):
            return (v << np.uint32(d)) | (v >> np.uint32(32 - d))

        x = [x0 + ks[0], x1 + ks[1]]

        def four_rounds(x, rots):
            for r in rots:
                x[0] = x[0] + x[1]
                x[1] = x[0] ^ rotl(x[1], r)
            return x

        ra, rb = (13, 15, 26, 6), (17, 29, 16, 24)
        x = four_rounds(x, ra); x[0] += ks[1]; x[1] += ks[2] + np.uint32(1)
        x = four_rounds(x, rb); x[0] += ks[2]; x[1] += ks[0] + np.uint32(2)
        x = four_rounds(x, ra); x[0] += ks[0]; x[1] += ks[1] + np.uint32(3)
        x = four_rounds(x, rb); x[0] += ks[1]; x[1] += ks[2] + np.uint32(4)
        x = four_rounds(x, ra); x[0] += ks[2]; x[1] += ks[0] + np.uint32(5)
        bits = x[0] ^ x[1]

        fb = (bits >> np.uint32(9)) | np.float32(1.0).view(np.uint32)
        f = fb.view(np.float32) - np.float32(1.0)
        tiny = np.float32(np.finfo(np.float32).tiny)
        u = np.maximum(tiny, f * (np.float32(1.0) - tiny) + tiny)
        return (-np.log(-np.log(u))).astype(np.float32)
    finally:
        np.seterr(**old)


_GUMBEL = _gumbel_table()

_mesh = plsc.VectorSubcoreMesh(
    core_axis_name="c", subcore_axis_name="s", num_cores=1)


@functools.partial(
    pl.kernel,
    out_type=jax.ShapeDtypeStruct((_L,), jnp.int32),
    mesh=_mesh,
    scratch_types=[
        pltpu.VMEM((_PER,), jnp.float32),
        pltpu.VMEM((_PER,), jnp.float32),
        pltpu.VMEM((_L,), jnp.float32),
        pltpu.VMEM((_L,), jnp.int32),
        pltpu.VMEM((_NT * _L,), jnp.float32),
        pltpu.VMEM_SHARED((_NT * _L,), jnp.float32),
        pltpu.SemaphoreType.DMA,
        pltpu.SemaphoreType.DMA,
        pltpu.SemaphoreType.DMA,
    ],
    compiler_params=pltpu.CompilerParams(
        needs_layout_passes=False, skip_device_barrier=True),
)
def _sc_sample(x_hbm, g_hbm, out_hbm, xv, gv, stage, stage_o, red, sh,
               sem, sem_g1, sem_g2):
    wid = lax.axis_index("s")
    is_last = wid == _NT - 1
    base = wid * _PER
    h1 = _PER // 2
    nv2 = jnp.where(is_last, _LAST // _L - h1 // _L, h1 // _L)

    g1_copy = pltpu.async_copy(g_hbm.at[pl.ds(base, h1)],
                               gv.at[pl.ds(0, h1)], sem_g1)

    @pl.when(is_last)
    def _():
        g2 = pltpu.async_copy(g_hbm.at[pl.ds(base + h1, _LAST - h1)],
                              gv.at[pl.ds(h1, _LAST - h1)], sem_g2)
        x2 = pltpu.async_copy(x_hbm.at[pl.ds(base + h1, _LAST - h1)],
                              xv.at[pl.ds(h1, _LAST - h1)], sem)
        pltpu.sync_copy(x_hbm.at[pl.ds(base, h1)], xv.at[pl.ds(0, h1)])
        del g2, x2

    @pl.when(jnp.logical_not(is_last))
    def _():
        g2 = pltpu.async_copy(g_hbm.at[pl.ds(base + h1, h1)],
                              gv.at[pl.ds(h1, h1)], sem_g2)
        x2 = pltpu.async_copy(x_hbm.at[pl.ds(base + h1, h1)],
                              xv.at[pl.ds(h1, h1)], sem)
        pltpu.sync_copy(x_hbm.at[pl.ds(base, h1)], xv.at[pl.ds(0, h1)])
        del g2, x2

    g1_copy.wait()
    lane = lax.iota(jnp.int32, _L)
    ninf = jnp.full((_L,), -jnp.inf, jnp.float32)

    def step(i, carry):
        m16, vb, ib = carry
        off = i * _L
        xs = xv[pl.ds(off, _L)]
        v = xs + gv[pl.ds(off, _L)]
        upd = v > vb
        return (jnp.maximum(m16, xs),
                jnp.where(upd, v, vb),
                jnp.where(upd, lane + (base + off), ib))

    carry0 = (ninf, ninf, jnp.full((_L,), 0, jnp.int32))
    carry1 = plsc.parallel_loop(0, h1 // _L, unroll=8, carry=carry0)(step)

    @pl.when(is_last)
    def _():
        pltpu.make_async_copy(x_hbm.at[pl.ds(base + h1, _LAST - h1)],
                              xv.at[pl.ds(h1, _LAST - h1)], sem).wait()
        pltpu.make_async_copy(g_hbm.at[pl.ds(base + h1, _LAST - h1)],
                              gv.at[pl.ds(h1, _LAST - h1)], sem_g2).wait()

    @pl.when(jnp.logical_not(is_last))
    def _():
        pltpu.make_async_copy(x_hbm.at[pl.ds(base + h1, h1)],
                              xv.at[pl.ds(h1, h1)], sem).wait()
        pltpu.make_async_copy(g_hbm.at[pl.ds(base + h1, h1)],
                              gv.at[pl.ds(h1, h1)], sem_g2).wait()

    def step2(i, carry):
        return step(i + h1 // _L, carry)

    m16, vb, ib = plsc.parallel_loop(0, nv2, unroll=8, carry=carry1)(step2)
    m_t = jnp.max(m16)
    best = jnp.max(vb)
    bidx = jnp.min(jnp.where(vb == best, ib, jnp.int32(2**31 - 1)))

    packed = jnp.where(lane == 0, best - m_t,
                       jnp.where(lane == 1, plsc.bitcast(
                           jnp.full((_L,), bidx, jnp.int32), jnp.float32),
                           jnp.full((_L,), m_t, jnp.float32)))
    stage[...] = packed
    pltpu.sync_copy(stage, sh.at[pl.ds(wid * _L, _L)])
    plsc.subcore_barrier()

    @pl.when(wid == 0)
    def _():
        pltpu.sync_copy(sh, red)

        def fbody(r, carry):
            cb, ci = carry
            row = red[pl.ds(r * _L, _L)]
            rowi = plsc.bitcast(row, jnp.int32)
            score = jnp.max(jnp.where(lane == 0, row, ninf)) \
                + jnp.max(jnp.where(lane == 2, row, ninf))
            bi = jnp.max(jnp.where(lane == 1, rowi,
                                   jnp.full((_L,), -2**31, jnp.int32)))
            take = score > cb
            return jnp.where(take, score, cb), jnp.where(take, bi, ci)

        _, fi = lax.fori_loop(0, _NT, fbody,
                              (jnp.float32(-jnp.inf), jnp.int32(0)))
        stage_o[...] = jnp.full((_L,), fi, jnp.int32)
        pltpu.sync_copy(stage_o, out_hbm)


def kernel(outputs):
    x = outputs.reshape(_VOCAB)
    g = jnp.asarray(_GUMBEL)
    winner = _sc_sample(x, g)
    return winner[:1].reshape(1, 1).astype(jnp.int64)


# speedup vs baseline: 1.3376x; 1.0180x over previous
import functools

import numpy as np

import jax
import jax.numpy as jnp
from jax import lax
from jax.experimental import pallas as pl
from jax.experimental.pallas import tpu as pltpu
from jax.experimental.pallas import tpu_sc as plsc

_VOCAB = 100000
_NT = 16
_PER = 6272
_LAST = _VOCAB - 15 * _PER
_L = 16


def _gumbel_table() -> np.ndarray:
    old = np.seterr(over="ignore")
    try:
        idx = np.arange(_VOCAB, dtype=np.uint64)
        x0 = (idx >> np.uint64(32)).astype(np.uint32)
        x1 = (idx & np.uint64(0xFFFFFFFF)).astype(np.uint32)
        k0, k1 = np.uint32(0), np.uint32(42)
        ks = [k0, k1, k0 ^ k1 ^ np.uint32(0x1BD11BDA)]

        def rotl(v, d):
            return (v << np.uint32(d)) | (v >> np.uint32(32 - d))

        x = [x0 + ks[0], x1 + ks[1]]

        def four_rounds(x, rots):
            for r in rots:
                x[0] = x[0] + x[1]
                x[1] = x[0] ^ rotl(x[1], r)
            return x

        ra, rb = (13, 15, 26, 6), (17, 29, 16, 24)
        x = four_rounds(x, ra); x[0] += ks[1]; x[1] += ks[2] + np.uint32(1)
        x = four_rounds(x, rb); x[0] += ks[2]; x[1] += ks[0] + np.uint32(2)
        x = four_rounds(x, ra); x[0] += ks[0]; x[1] += ks[1] + np.uint32(3)
        x = four_rounds(x, rb); x[0] += ks[1]; x[1] += ks[2] + np.uint32(4)
        x = four_rounds(x, ra); x[0] += ks[2]; x[1] += ks[0] + np.uint32(5)
        bits = x[0] ^ x[1]

        fb = (bits >> np.uint32(9)) | np.float32(1.0).view(np.uint32)
        f = fb.view(np.float32) - np.float32(1.0)
        tiny = np.float32(np.finfo(np.float32).tiny)
        u = np.maximum(tiny, f * (np.float32(1.0) - tiny) + tiny)
        return (-np.log(-np.log(u))).astype(np.float32)
    finally:
        np.seterr(**old)


_GUMBEL = _gumbel_table()

_mesh = plsc.VectorSubcoreMesh(
    core_axis_name="c", subcore_axis_name="s", num_cores=1)


@functools.partial(
    pl.kernel,
    out_type=jax.ShapeDtypeStruct((_L,), jnp.int32),
    mesh=_mesh,
    scratch_types=[
        pltpu.VMEM((_PER,), jnp.float32),
        pltpu.VMEM((_PER,), jnp.float32),
        pltpu.VMEM((_L,), jnp.float32),
        pltpu.VMEM((_L,), jnp.int32),
        pltpu.VMEM((_NT * _L,), jnp.float32),
        pltpu.VMEM_SHARED((_NT * _L,), jnp.float32),
        pltpu.SemaphoreType.DMA,
    ],
    compiler_params=pltpu.CompilerParams(
        needs_layout_passes=False, skip_device_barrier=True),
)
def _sc_sample(x_hbm, g_hbm, out_hbm, xv, gv, stage, stage_o, red, sh, sem):
    wid = lax.axis_index("s")
    is_last = wid == _NT - 1
    base = wid * _PER
    nv = jnp.where(is_last, _LAST // _L, _PER // _L)

    @pl.when(is_last)
    def _():
        g_copy = pltpu.async_copy(g_hbm.at[pl.ds(base, _LAST)],
                                  gv.at[pl.ds(0, _LAST)], sem)
        pltpu.sync_copy(x_hbm.at[pl.ds(base, _LAST)], xv.at[pl.ds(0, _LAST)])
        g_copy.wait()

    @pl.when(jnp.logical_not(is_last))
    def _():
        g_copy = pltpu.async_copy(g_hbm.at[pl.ds(base, _PER)], gv, sem)
        pltpu.sync_copy(x_hbm.at[pl.ds(base, _PER)], xv)
        g_copy.wait()

    lane = lax.iota(jnp.int32, _L)
    ninf = jnp.full((_L,), -jnp.inf, jnp.float32)

    @plsc.parallel_loop(
        0, nv, unroll=8,
        carry=(ninf, ninf, jnp.full((_L,), 0, jnp.int32)))
    def _sample(i, carry):
        m16, vb, ib = carry
        off = i * _L
        xs = xv[pl.ds(off, _L)]
        v = xs + gv[pl.ds(off, _L)]
        upd = v > vb
        return (jnp.maximum(m16, xs),
                jnp.where(upd, v, vb),
                jnp.where(upd, lane + (base + off), ib))

    m16, vb, ib = _sample
    m_t = jnp.max(m16)
    best = jnp.max(vb)
    bidx = jnp.min(jnp.where(vb == best, ib, jnp.int32(2**31 - 1)))

    packed = jnp.where(lane == 0, best - m_t,
                       jnp.where(lane == 1, plsc.bitcast(
                           jnp.full((_L,), bidx, jnp.int32), jnp.float32),
                           jnp.full((_L,), m_t, jnp.float32)))
    stage[...] = packed
    pltpu.sync_copy(stage, sh.at[pl.ds(wid * _L, _L)])
    plsc.subcore_barrier()

    @pl.when(wid == 0)
    def _():
        pltpu.sync_copy(sh, red)

        def fbody(r, carry):
            cb, ci = carry
            row = red[pl.ds(r * _L, _L)]
            rowi = plsc.bitcast(row, jnp.int32)
            score = jnp.max(jnp.where(lane == 0, row, ninf)) \
                + jnp.max(jnp.where(lane == 2, row, ninf))
            bi = jnp.max(jnp.where(lane == 1, rowi,
                                   jnp.full((_L,), -2**31, jnp.int32)))
            take = score > cb
            return jnp.where(take, score, cb), jnp.where(take, bi, ci)

        _, fi = lax.fori_loop(0, _NT, fbody,
                              (jnp.float32(-jnp.inf), jnp.int32(0)))
        stage_o[...] = jnp.full((_L,), fi, jnp.int32)
        pltpu.sync_copy(stage_o, out_hbm)


def kernel(outputs):
    x = outputs.reshape(_VOCAB)
    g = jnp.asarray(_GUMBEL)
    winner = _sc_sample(x, g)
    return winner[:1].reshape(1, 1).astype(jnp.int64)
